# baseline (speedup 1.0000x reference)
import functools

import jax
import jax.numpy as jnp
from jax import lax
from jax.experimental import pallas as pl
from jax.experimental.pallas import tpu as pltpu

L = 2048
DIM = 1024
DIN = 2048
DSTATE = 16
DTRANK = 64
DCONV = 4
NEXP = 8
FFI = 2048
LC = 64
RC = 256

_F32 = jnp.float32
_NT = (((1,), (1,)), ((), ()))


def _silu(v):
    return v * jax.nn.sigmoid(v)


def _softplus(v):
    return jnp.maximum(v, 0.0) + jnp.log1p(jnp.exp(-jnp.abs(v)))


def _ntdot(a, b):
    return lax.dot_general(a, b, _NT, preferred_element_type=_F32)


def _mamba_body(x_ref, w_in_ref, conv_w_ref, conv_b_ref, wx_ref,
                w_dt_ref, b_dt_ref, alog_ref, dd_ref, w_out_ref,
                out_ref, tail_ref, state_ref, da_s, dbu_s, st_s):
    c = pl.program_id(0)

    @pl.when(c == 0)
    def _():
        tail_ref[...] = jnp.zeros_like(tail_ref)
        state_ref[...] = jnp.zeros_like(state_ref)

    xch = x_ref[...]
    xz = _ntdot(xch, w_in_ref[...])
    xc = xz[:, :DIN]
    res = xz[:, DIN:]

    ext = jnp.concatenate([tail_ref[...], xc], axis=0)
    tail_ref[...] = xc[LC - (DCONV - 1):, :]
    conv = conv_b_ref[...]
    for k in range(DCONV):
        conv = conv + ext[k:k + LC, :] * conv_w_ref[k:k + 1, :]
    xcs = _silu(conv)

    x_dbl = _ntdot(xcs, wx_ref[...])
    delta_r = x_dbl[:, :DTRANK]
    bm = x_dbl[:, DTRANK:DTRANK + DSTATE]
    cm = x_dbl[:, DTRANK + DSTATE:]
    delta = _softplus(_ntdot(delta_r, w_dt_ref[...]) + b_dt_ref[...])
    u = delta * xcs

    a2 = -jnp.exp(alog_ref[...])
    da_s[...] = jnp.exp(delta[:, None, :] * a2[None, :, :])
    dbu_s[...] = bm[:, :, None] * u[:, None, :]

    def step(l, _):
        st = (da_s[pl.ds(l, 1)][0] * state_ref[...]
              + dbu_s[pl.ds(l, 1)][0])
        state_ref[...] = st
        st_s[pl.ds(l, 1)] = st[None]
        return 0

    lax.fori_loop(0, LC, step, 0, unroll=False)

    y = jnp.sum(st_s[...] * cm[:, :, None], axis=1)
    y = y + xcs * dd_ref[...]
    y = y * _silu(res)
    out_ref[...] = _ntdot(y, w_out_ref[...]) + xch


def _mamba_block(h, bp):
    conv_w_t = bp['conv_w'].T
    conv_b = bp['conv_b'].reshape(1, DIN)
    b_dt = bp['b_dt'].reshape(1, DIN)
    alog_t = bp['A_log'].T
    dd = bp['D'].reshape(1, DIN)

    grid = L // LC
    full = lambda shape: pl.BlockSpec(shape, lambda c: (0,) * len(shape))
    return pl.pallas_call(
        _mamba_body,
        grid=(grid,),
        in_specs=[
            pl.BlockSpec((LC, DIM), lambda c: (c, 0)),
            full((2 * DIN, DIM)),
            full((DCONV, DIN)),
            full((1, DIN)),
            full((DTRANK + 2 * DSTATE, DIN)),
            full((DIN, DTRANK)),
            full((1, DIN)),
            full((DSTATE, DIN)),
            full((1, DIN)),
            full((DIM, DIN)),
        ],
        out_specs=pl.BlockSpec((LC, DIM), lambda c: (c, 0)),
        out_shape=jax.ShapeDtypeStruct((L, DIM), _F32),
        scratch_shapes=[
            pltpu.VMEM((DCONV - 1, DIN), _F32),
            pltpu.VMEM((DSTATE, DIN), _F32),
            pltpu.VMEM((LC, DSTATE, DIN), _F32),
            pltpu.VMEM((LC, DSTATE, DIN), _F32),
            pltpu.VMEM((LC, DSTATE, DIN), _F32),
        ],
    )(h, bp['W_in'], conv_w_t, conv_b, bp['W_x'], bp['W_dt'], b_dt,
      alog_t, dd, bp['W_out'])


def _top2_weight(h, wgate, e):
    scores = _ntdot(h, wgate)
    ii = lax.broadcasted_iota(jnp.int32, scores.shape, 1)
    m1 = jnp.max(scores, axis=-1, keepdims=True)
    a1 = jnp.min(jnp.where(scores == m1, ii, NEXP), axis=-1, keepdims=True)
    s2 = jnp.where(ii == a1, -jnp.inf, scores)
    m2 = jnp.max(s2, axis=-1, keepdims=True)
    a2 = jnp.min(jnp.where(s2 == m2, ii, NEXP), axis=-1, keepdims=True)
    e2 = jnp.exp(m2 - m1)
    w1 = 1.0 / (1.0 + e2)
    w2 = 1.0 - w1
    return jnp.where(a1 == e, w1, 0.0) + jnp.where(a2 == e, w2, 0.0)


KK = 8
FB = FFI // KK


def _moe_body(h_ref, wgate_ref, *refs):
    wg_refs = refs[0:NEXP]
    wu_refs = refs[NEXP:2 * NEXP]
    wd_refs = refs[2 * NEXP:3 * NEXP]
    out_ref = refs[3 * NEXP]
    we_s = refs[3 * NEXP + 1]

    e = pl.program_id(1)
    kk = pl.program_id(2)
    h = h_ref[...]

    @pl.when(kk == 0)
    def _():
        we_s[...] = _top2_weight(h, wgate_ref[...], e)

    @pl.when((e == 0) & (kk == 0))
    def _():
        out_ref[...] = h

    for j in range(NEXP):
        @pl.when(e == j)
        def _(j=j):
            gate = _silu(_ntdot(h, wg_refs[j][...]))
            up = _ntdot(h, wu_refs[j][...])
            part = _ntdot(gate * up, wd_refs[j][...])
            out_ref[...] = out_ref[...] + we_s[...] * part


def _moe_block(h, mp):
    wg_spec = lambda j: pl.BlockSpec(
        (FB, DIM), functools.partial(
            lambda j_, r, e, kk: (jnp.where(e == j_, kk, 0), 0), j))
    wd_spec = lambda j: pl.BlockSpec(
        (DIM, FB), functools.partial(
            lambda j_, r, e, kk: (0, jnp.where(e == j_, kk, 0)), j))
    in_specs = (
        [pl.BlockSpec((RC, DIM), lambda r, e, kk: (r, 0)),
         pl.BlockSpec((NEXP, DIM), lambda r, e, kk: (0, 0))]
        + [wg_spec(j) for j in range(NEXP)]
        + [wg_spec(j) for j in range(NEXP)]
        + [wd_spec(j) for j in range(NEXP)]
    )
    eps = mp['experts']
    args = ([h, mp['W_gate']]
            + [eps[j]['Wg'] for j in range(NEXP)]
            + [eps[j]['Wu'] for j in range(NEXP)]
            + [eps[j]['Wd'] for j in range(NEXP)])
    return pl.pallas_call(
        _moe_body,
        grid=(L // RC, NEXP, KK),
        in_specs=in_specs,
        out_specs=pl.BlockSpec((RC, DIM), lambda r, e, kk: (r, 0)),
        out_shape=jax.ShapeDtypeStruct((L, DIM), _F32),
        scratch_shapes=[pltpu.VMEM((RC, 1), _F32)],
    )(*args)


def _head_body(h_ref, w_ref, out_ref):
    out_ref[...] = jax.nn.sigmoid(_ntdot(h_ref[...], w_ref[...]))


def _head(h, w_head):
    return pl.pallas_call(
        _head_body,
        grid=(L // RC,),
        in_specs=[
            pl.BlockSpec((RC, DIM), lambda r: (r, 0)),
            pl.BlockSpec((DIM, DIM), lambda r: (0, 0)),
        ],
        out_specs=pl.BlockSpec((RC, DIM), lambda r: (r, 0)),
        out_shape=jax.ShapeDtypeStruct((L, DIM), _F32),
    )(h, w_head)


def kernel(x, params):
    h = x.reshape(L, DIM)
    for i in range(len(params['blocks'])):
        h = _mamba_block(h, params['blocks'][i])
        h = _moe_block(h, params['moes'][i])
    h = _head(h, params['W_head'])
    return h.reshape(x.shape)


# speedup vs baseline: 1.8017x; 1.8017x over previous
import functools

import jax
import jax.numpy as jnp
from jax import lax
from jax.experimental import pallas as pl
from jax.experimental.pallas import tpu as pltpu

L = 2048
DIM = 1024
DIN = 2048
DSTATE = 16
DTRANK = 64
DCONV = 4
NEXP = 8
FFI = 2048
LC = 64
RC = 256

_F32 = jnp.float32
_NT = (((1,), (1,)), ((), ()))


def _silu(v):
    return v * jax.nn.sigmoid(v)


def _softplus(v):
    return jnp.maximum(v, 0.0) + jnp.log1p(jnp.exp(-jnp.abs(v)))


def _ntdot(a, b):
    return lax.dot_general(a, b, _NT, preferred_element_type=_F32)


def _mamba_body(x_ref, w_in_ref, conv_w_ref, conv_b_ref, wx_ref,
                w_dt_ref, b_dt_ref, alog_ref, dd_ref, w_out_ref,
                out_ref, tail_ref, state_ref, da_s, dbu_s, st_s):
    c = pl.program_id(0)

    @pl.when(c == 0)
    def _():
        tail_ref[...] = jnp.zeros_like(tail_ref)
        state_ref[...] = jnp.zeros_like(state_ref)

    xch = x_ref[...]
    xz = _ntdot(xch, w_in_ref[...])
    xc = xz[:, :DIN]
    res = xz[:, DIN:]

    ext = jnp.concatenate([tail_ref[...], xc], axis=0)
    tail_ref[...] = xc[LC - (DCONV - 1):, :]
    conv = conv_b_ref[...]
    for k in range(DCONV):
        conv = conv + ext[k:k + LC, :] * conv_w_ref[k:k + 1, :]
    xcs = _silu(conv)

    x_dbl = _ntdot(xcs, wx_ref[...])
    delta_r = x_dbl[:, :DTRANK]
    bm = x_dbl[:, DTRANK:DTRANK + DSTATE]
    cm = x_dbl[:, DTRANK + DSTATE:]
    delta = _softplus(_ntdot(delta_r, w_dt_ref[...]) + b_dt_ref[...])
    u = delta * xcs

    a2 = -jnp.exp(alog_ref[...])
    da_s[...] = jnp.exp(delta[:, None, :] * a2[None, :, :])
    dbu_s[...] = bm[:, :, None] * u[:, None, :]

    def step(l, _):
        st = (da_s[pl.ds(l, 1)][0] * state_ref[...]
              + dbu_s[pl.ds(l, 1)][0])
        state_ref[...] = st
        st_s[pl.ds(l, 1)] = st[None]
        return 0

    lax.fori_loop(0, LC, step, 0, unroll=False)

    y = jnp.sum(st_s[...] * cm[:, :, None], axis=1)
    y = y + xcs * dd_ref[...]
    y = y * _silu(res)
    out_ref[...] = _ntdot(y, w_out_ref[...]) + xch


def _mamba_block(h, bp):
    conv_w_t = bp['conv_w'].T
    conv_b = bp['conv_b'].reshape(1, DIN)
    b_dt = bp['b_dt'].reshape(1, DIN)
    alog_t = bp['A_log'].T
    dd = bp['D'].reshape(1, DIN)

    grid = L // LC
    full = lambda shape: pl.BlockSpec(shape, lambda c: (0,) * len(shape))
    return pl.pallas_call(
        _mamba_body,
        grid=(grid,),
        in_specs=[
            pl.BlockSpec((LC, DIM), lambda c: (c, 0)),
            full((2 * DIN, DIM)),
            full((DCONV, DIN)),
            full((1, DIN)),
            full((DTRANK + 2 * DSTATE, DIN)),
            full((DIN, DTRANK)),
            full((1, DIN)),
            full((DSTATE, DIN)),
            full((1, DIN)),
            full((DIM, DIN)),
        ],
        out_specs=pl.BlockSpec((LC, DIM), lambda c: (c, 0)),
        out_shape=jax.ShapeDtypeStruct((L, DIM), _F32),
        scratch_shapes=[
            pltpu.VMEM((DCONV - 1, DIN), _F32),
            pltpu.VMEM((DSTATE, DIN), _F32),
            pltpu.VMEM((LC, DSTATE, DIN), _F32),
            pltpu.VMEM((LC, DSTATE, DIN), _F32),
            pltpu.VMEM((LC, DSTATE, DIN), _F32),
        ],
    )(h, bp['W_in'], conv_w_t, conv_b, bp['W_x'], bp['W_dt'], b_dt,
      alog_t, dd, bp['W_out'])


def _top2_weight(h, wgate, e):
    scores = _ntdot(h, wgate)
    ii = lax.broadcasted_iota(jnp.int32, scores.shape, 1)
    m1 = jnp.max(scores, axis=-1, keepdims=True)
    a1 = jnp.min(jnp.where(scores == m1, ii, NEXP), axis=-1, keepdims=True)
    s2 = jnp.where(ii == a1, -jnp.inf, scores)
    m2 = jnp.max(s2, axis=-1, keepdims=True)
    a2 = jnp.min(jnp.where(s2 == m2, ii, NEXP), axis=-1, keepdims=True)
    e2 = jnp.exp(m2 - m1)
    w1 = 1.0 / (1.0 + e2)
    w2 = 1.0 - w1
    return jnp.where(a1 == e, w1, 0.0) + jnp.where(a2 == e, w2, 0.0)


EPC = 2


def _moe_pair_body(e0, h_ref, acc_ref, wgate_ref, wg0_ref, wu0_ref, wd0_ref,
                   wg1_ref, wu1_ref, wd1_ref, out_ref):
    h = h_ref[...]
    out = acc_ref[...]
    for d, (wg, wu, wd) in enumerate(((wg0_ref, wu0_ref, wd0_ref),
                                      (wg1_ref, wu1_ref, wd1_ref))):
        we = _top2_weight(h, wgate_ref[...], e0 + d)
        gate = _silu(_ntdot(h, wg[...]))
        up = _ntdot(h, wu[...])
        out = out + we * _ntdot(gate * up, wd[...])
    out_ref[...] = out


def _moe_block(h, mp):
    acc = h
    for e0 in range(0, NEXP, EPC):
        ea, eb = mp['experts'][e0], mp['experts'][e0 + 1]
        acc = pl.pallas_call(
            functools.partial(_moe_pair_body, e0),
            grid=(L // RC,),
            in_specs=[
                pl.BlockSpec((RC, DIM), lambda r: (r, 0)),
                pl.BlockSpec((RC, DIM), lambda r: (r, 0)),
                pl.BlockSpec((NEXP, DIM), lambda r: (0, 0)),
                pl.BlockSpec((FFI, DIM), lambda r: (0, 0)),
                pl.BlockSpec((FFI, DIM), lambda r: (0, 0)),
                pl.BlockSpec((DIM, FFI), lambda r: (0, 0)),
                pl.BlockSpec((FFI, DIM), lambda r: (0, 0)),
                pl.BlockSpec((FFI, DIM), lambda r: (0, 0)),
                pl.BlockSpec((DIM, FFI), lambda r: (0, 0)),
            ],
            out_specs=pl.BlockSpec((RC, DIM), lambda r: (r, 0)),
            out_shape=jax.ShapeDtypeStruct((L, DIM), _F32),
        )(h, acc, mp['W_gate'], ea['Wg'], ea['Wu'], ea['Wd'],
          eb['Wg'], eb['Wu'], eb['Wd'])
    return acc


def _head_body(h_ref, w_ref, out_ref):
    out_ref[...] = jax.nn.sigmoid(_ntdot(h_ref[...], w_ref[...]))


def _head(h, w_head):
    return pl.pallas_call(
        _head_body,
        grid=(L // RC,),
        in_specs=[
            pl.BlockSpec((RC, DIM), lambda r: (r, 0)),
            pl.BlockSpec((DIM, DIM), lambda r: (0, 0)),
        ],
        out_specs=pl.BlockSpec((RC, DIM), lambda r: (r, 0)),
        out_shape=jax.ShapeDtypeStruct((L, DIM), _F32),
    )(h, w_head)


def kernel(x, params):
    h = x.reshape(L, DIM)
    for i in range(len(params['blocks'])):
        h = _mamba_block(h, params['blocks'][i])
        h = _moe_block(h, params['moes'][i])
    h = _head(h, params['W_head'])
    return h.reshape(x.shape)


# speedup vs baseline: 1.8507x; 1.0272x over previous
import functools

import jax
import jax.numpy as jnp
from jax import lax
from jax.experimental import pallas as pl
from jax.experimental.pallas import tpu as pltpu

L = 2048
DIM = 1024
DIN = 2048
DSTATE = 16
DTRANK = 64
DCONV = 4
NEXP = 8
FFI = 2048
LC = 64
RC = 256

_F32 = jnp.float32
_NT = (((1,), (1,)), ((), ()))


def _silu(v):
    return v * jax.nn.sigmoid(v)


def _softplus(v):
    return jnp.maximum(v, 0.0) + jnp.log1p(jnp.exp(-jnp.abs(v)))


def _ntdot(a, b):
    return lax.dot_general(a, b, _NT, preferred_element_type=_F32)


def _mamba_body(x_ref, w_in_ref, conv_w_ref, conv_b_ref, wx_ref,
                w_dt_ref, b_dt_ref, alog_ref, dd_ref, w_out_ref,
                out_ref, tail_ref, state_ref, da_s, dbu_s, st_s):
    c = pl.program_id(0)

    @pl.when(c == 0)
    def _():
        tail_ref[...] = jnp.zeros_like(tail_ref)
        state_ref[...] = jnp.zeros_like(state_ref)

    xch = x_ref[...]
    xz = _ntdot(xch, w_in_ref[...])
    xc = xz[:, :DIN]
    res = xz[:, DIN:]

    ext = jnp.concatenate([tail_ref[...], xc], axis=0)
    tail_ref[...] = xc[LC - (DCONV - 1):, :]
    conv = conv_b_ref[...]
    for k in range(DCONV):
        conv = conv + ext[k:k + LC, :] * conv_w_ref[k:k + 1, :]
    xcs = _silu(conv)

    x_dbl = _ntdot(xcs, wx_ref[...])
    delta_r = x_dbl[:, :DTRANK]
    bm = x_dbl[:, DTRANK:DTRANK + DSTATE]
    cm = x_dbl[:, DTRANK + DSTATE:]
    delta = _softplus(_ntdot(delta_r, w_dt_ref[...]) + b_dt_ref[...])
    u = delta * xcs

    a2 = -jnp.exp(alog_ref[...])
    da_s[...] = jnp.exp(delta[:, None, :] * a2[None, :, :])
    dbu_s[...] = bm[:, :, None] * u[:, None, :]

    def step(l, _):
        st = (da_s[pl.ds(l, 1)][0] * state_ref[...]
              + dbu_s[pl.ds(l, 1)][0])
        state_ref[...] = st
        st_s[pl.ds(l, 1)] = st[None]
        return 0

    lax.fori_loop(0, LC, step, 0, unroll=4)

    y = jnp.sum(st_s[...] * cm[:, :, None], axis=1)
    y = y + xcs * dd_ref[...]
    y = y * _silu(res)
    out_ref[...] = _ntdot(y, w_out_ref[...]) + xch


def _mamba_block(h, bp):
    conv_w_t = bp['conv_w'].T
    conv_b = bp['conv_b'].reshape(1, DIN)
    b_dt = bp['b_dt'].reshape(1, DIN)
    alog_t = bp['A_log'].T
    dd = bp['D'].reshape(1, DIN)

    grid = L // LC
    full = lambda shape: pl.BlockSpec(shape, lambda c: (0,) * len(shape))
    return pl.pallas_call(
        _mamba_body,
        grid=(grid,),
        in_specs=[
            pl.BlockSpec((LC, DIM), lambda c: (c, 0)),
            full((2 * DIN, DIM)),
            full((DCONV, DIN)),
            full((1, DIN)),
            full((DTRANK + 2 * DSTATE, DIN)),
            full((DIN, DTRANK)),
            full((1, DIN)),
            full((DSTATE, DIN)),
            full((1, DIN)),
            full((DIM, DIN)),
        ],
        out_specs=pl.BlockSpec((LC, DIM), lambda c: (c, 0)),
        out_shape=jax.ShapeDtypeStruct((L, DIM), _F32),
        scratch_shapes=[
            pltpu.VMEM((DCONV - 1, DIN), _F32),
            pltpu.VMEM((DSTATE, DIN), _F32),
            pltpu.VMEM((LC, DSTATE, DIN), _F32),
            pltpu.VMEM((LC, DSTATE, DIN), _F32),
            pltpu.VMEM((LC, DSTATE, DIN), _F32),
        ],
    )(h, bp['W_in'], conv_w_t, conv_b, bp['W_x'], bp['W_dt'], b_dt,
      alog_t, dd, bp['W_out'])


def _top2_weight(h, wgate, e):
    scores = _ntdot(h, wgate)
    ii = lax.broadcasted_iota(jnp.int32, scores.shape, 1)
    m1 = jnp.max(scores, axis=-1, keepdims=True)
    a1 = jnp.min(jnp.where(scores == m1, ii, NEXP), axis=-1, keepdims=True)
    s2 = jnp.where(ii == a1, -jnp.inf, scores)
    m2 = jnp.max(s2, axis=-1, keepdims=True)
    a2 = jnp.min(jnp.where(s2 == m2, ii, NEXP), axis=-1, keepdims=True)
    e2 = jnp.exp(m2 - m1)
    w1 = 1.0 / (1.0 + e2)
    w2 = 1.0 - w1
    return jnp.where(a1 == e, w1, 0.0) + jnp.where(a2 == e, w2, 0.0)


EPC = 2


def _moe_pair_body(e0, h_ref, acc_ref, wgate_ref, wg0_ref, wu0_ref, wd0_ref,
                   wg1_ref, wu1_ref, wd1_ref, out_ref):
    h = h_ref[...]
    out = acc_ref[...]
    for d, (wg, wu, wd) in enumerate(((wg0_ref, wu0_ref, wd0_ref),
                                      (wg1_ref, wu1_ref, wd1_ref))):
        we = _top2_weight(h, wgate_ref[...], e0 + d)
        gate = _silu(_ntdot(h, wg[...]))
        up = _ntdot(h, wu[...])
        out = out + we * _ntdot(gate * up, wd[...])
    out_ref[...] = out


def _moe_block(h, mp):
    acc = h
    for e0 in range(0, NEXP, EPC):
        ea, eb = mp['experts'][e0], mp['experts'][e0 + 1]
        acc = pl.pallas_call(
            functools.partial(_moe_pair_body, e0),
            grid=(L // RC,),
            in_specs=[
                pl.BlockSpec((RC, DIM), lambda r: (r, 0)),
                pl.BlockSpec((RC, DIM), lambda r: (r, 0)),
                pl.BlockSpec((NEXP, DIM), lambda r: (0, 0)),
                pl.BlockSpec((FFI, DIM), lambda r: (0, 0)),
                pl.BlockSpec((FFI, DIM), lambda r: (0, 0)),
                pl.BlockSpec((DIM, FFI), lambda r: (0, 0)),
                pl.BlockSpec((FFI, DIM), lambda r: (0, 0)),
                pl.BlockSpec((FFI, DIM), lambda r: (0, 0)),
                pl.BlockSpec((DIM, FFI), lambda r: (0, 0)),
            ],
            out_specs=pl.BlockSpec((RC, DIM), lambda r: (r, 0)),
            out_shape=jax.ShapeDtypeStruct((L, DIM), _F32),
        )(h, acc, mp['W_gate'], ea['Wg'], ea['Wu'], ea['Wd'],
          eb['Wg'], eb['Wu'], eb['Wd'])
    return acc


def _head_body(h_ref, w_ref, out_ref):
    out_ref[...] = jax.nn.sigmoid(_ntdot(h_ref[...], w_ref[...]))


def _head(h, w_head):
    return pl.pallas_call(
        _head_body,
        grid=(L // RC,),
        in_specs=[
            pl.BlockSpec((RC, DIM), lambda r: (r, 0)),
            pl.BlockSpec((DIM, DIM), lambda r: (0, 0)),
        ],
        out_specs=pl.BlockSpec((RC, DIM), lambda r: (r, 0)),
        out_shape=jax.ShapeDtypeStruct((L, DIM), _F32),
    )(h, w_head)


def kernel(x, params):
    h = x.reshape(L, DIM)
    for i in range(len(params['blocks'])):
        h = _mamba_block(h, params['blocks'][i])
        h = _moe_block(h, params['moes'][i])
    h = _head(h, params['W_head'])
    return h.reshape(x.shape)


# speedup vs baseline: 1.8566x; 1.0032x over previous
import functools

import jax
import jax.numpy as jnp
from jax import lax
from jax.experimental import pallas as pl
from jax.experimental.pallas import tpu as pltpu

L = 2048
DIM = 1024
DIN = 2048
DSTATE = 16
DTRANK = 64
DCONV = 4
NEXP = 8
FFI = 2048
LC = 64
RC = 256

_F32 = jnp.float32
_NT = (((1,), (1,)), ((), ()))


def _silu(v):
    return v * jax.nn.sigmoid(v)


def _softplus(v):
    return jnp.maximum(v, 0.0) + jnp.log1p(jnp.exp(-jnp.abs(v)))


def _ntdot(a, b):
    return lax.dot_general(a, b, _NT, preferred_element_type=_F32)


def _mamba_body(x_ref, w_in_ref, conv_w_ref, conv_b_ref, wx_ref,
                w_dt_ref, b_dt_ref, alog_ref, dd_ref, w_out_ref,
                out_ref, tail_ref, state_ref, da_s, dbu_s, st_s):
    c = pl.program_id(0)

    @pl.when(c == 0)
    def _():
        tail_ref[...] = jnp.zeros_like(tail_ref)
        state_ref[...] = jnp.zeros_like(state_ref)

    xch = x_ref[...]
    xz = _ntdot(xch, w_in_ref[...])
    xc = xz[:, :DIN]
    res = xz[:, DIN:]

    ext = jnp.concatenate([tail_ref[...], xc], axis=0)
    tail_ref[...] = xc[LC - (DCONV - 1):, :]
    conv = conv_b_ref[...]
    for k in range(DCONV):
        conv = conv + ext[k:k + LC, :] * conv_w_ref[k:k + 1, :]
    xcs = _silu(conv)

    x_dbl = _ntdot(xcs, wx_ref[...])
    delta_r = x_dbl[:, :DTRANK]
    bm = x_dbl[:, DTRANK:DTRANK + DSTATE]
    cm = x_dbl[:, DTRANK + DSTATE:]
    delta = _softplus(_ntdot(delta_r, w_dt_ref[...]) + b_dt_ref[...])
    u = delta * xcs

    a2 = -jnp.exp(alog_ref[...])
    da_s[...] = jnp.exp(delta[:, None, :] * a2[None, :, :])
    dbu_s[...] = bm[:, :, None] * u[:, None, :]

    def step(l, _):
        st = (da_s[pl.ds(l, 1)][0] * state_ref[...]
              + dbu_s[pl.ds(l, 1)][0])
        state_ref[...] = st
        st_s[pl.ds(l, 1)] = st[None]
        return 0

    lax.fori_loop(0, LC, step, 0, unroll=8)

    y = jnp.sum(st_s[...] * cm[:, :, None], axis=1)
    y = y + xcs * dd_ref[...]
    y = y * _silu(res)
    out_ref[...] = _ntdot(y, w_out_ref[...]) + xch


def _mamba_block(h, bp):
    conv_w_t = bp['conv_w'].T
    conv_b = bp['conv_b'].reshape(1, DIN)
    b_dt = bp['b_dt'].reshape(1, DIN)
    alog_t = bp['A_log'].T
    dd = bp['D'].reshape(1, DIN)

    grid = L // LC
    full = lambda shape: pl.BlockSpec(shape, lambda c: (0,) * len(shape))
    return pl.pallas_call(
        _mamba_body,
        grid=(grid,),
        in_specs=[
            pl.BlockSpec((LC, DIM), lambda c: (c, 0)),
            full((2 * DIN, DIM)),
            full((DCONV, DIN)),
            full((1, DIN)),
            full((DTRANK + 2 * DSTATE, DIN)),
            full((DIN, DTRANK)),
            full((1, DIN)),
            full((DSTATE, DIN)),
            full((1, DIN)),
            full((DIM, DIN)),
        ],
        out_specs=pl.BlockSpec((LC, DIM), lambda c: (c, 0)),
        out_shape=jax.ShapeDtypeStruct((L, DIM), _F32),
        scratch_shapes=[
            pltpu.VMEM((DCONV - 1, DIN), _F32),
            pltpu.VMEM((DSTATE, DIN), _F32),
            pltpu.VMEM((LC, DSTATE, DIN), _F32),
            pltpu.VMEM((LC, DSTATE, DIN), _F32),
            pltpu.VMEM((LC, DSTATE, DIN), _F32),
        ],
    )(h, bp['W_in'], conv_w_t, conv_b, bp['W_x'], bp['W_dt'], b_dt,
      alog_t, dd, bp['W_out'])


def _top2_weight(h, wgate, e):
    scores = _ntdot(h, wgate)
    ii = lax.broadcasted_iota(jnp.int32, scores.shape, 1)
    m1 = jnp.max(scores, axis=-1, keepdims=True)
    a1 = jnp.min(jnp.where(scores == m1, ii, NEXP), axis=-1, keepdims=True)
    s2 = jnp.where(ii == a1, -jnp.inf, scores)
    m2 = jnp.max(s2, axis=-1, keepdims=True)
    a2 = jnp.min(jnp.where(s2 == m2, ii, NEXP), axis=-1, keepdims=True)
    e2 = jnp.exp(m2 - m1)
    w1 = 1.0 / (1.0 + e2)
    w2 = 1.0 - w1
    return jnp.where(a1 == e, w1, 0.0) + jnp.where(a2 == e, w2, 0.0)


EPC = 2


def _moe_pair_body(e0, h_ref, acc_ref, wgate_ref, wg0_ref, wu0_ref, wd0_ref,
                   wg1_ref, wu1_ref, wd1_ref, out_ref):
    h = h_ref[...]
    out = acc_ref[...]
    for d, (wg, wu, wd) in enumerate(((wg0_ref, wu0_ref, wd0_ref),
                                      (wg1_ref, wu1_ref, wd1_ref))):
        we = _top2_weight(h, wgate_ref[...], e0 + d)
        gate = _silu(_ntdot(h, wg[...]))
        up = _ntdot(h, wu[...])
        out = out + we * _ntdot(gate * up, wd[...])
    out_ref[...] = out


def _moe_block(h, mp):
    acc = h
    for e0 in range(0, NEXP, EPC):
        ea, eb = mp['experts'][e0], mp['experts'][e0 + 1]
        acc = pl.pallas_call(
            functools.partial(_moe_pair_body, e0),
            grid=(L // RC,),
            in_specs=[
                pl.BlockSpec((RC, DIM), lambda r: (r, 0)),
                pl.BlockSpec((RC, DIM), lambda r: (r, 0)),
                pl.BlockSpec((NEXP, DIM), lambda r: (0, 0)),
                pl.BlockSpec((FFI, DIM), lambda r: (0, 0)),
                pl.BlockSpec((FFI, DIM), lambda r: (0, 0)),
                pl.BlockSpec((DIM, FFI), lambda r: (0, 0)),
                pl.BlockSpec((FFI, DIM), lambda r: (0, 0)),
                pl.BlockSpec((FFI, DIM), lambda r: (0, 0)),
                pl.BlockSpec((DIM, FFI), lambda r: (0, 0)),
            ],
            out_specs=pl.BlockSpec((RC, DIM), lambda r: (r, 0)),
            out_shape=jax.ShapeDtypeStruct((L, DIM), _F32),
        )(h, acc, mp['W_gate'], ea['Wg'], ea['Wu'], ea['Wd'],
          eb['Wg'], eb['Wu'], eb['Wd'])
    return acc


def _head_body(h_ref, w_ref, out_ref):
    out_ref[...] = jax.nn.sigmoid(_ntdot(h_ref[...], w_ref[...]))


def _head(h, w_head):
    return pl.pallas_call(
        _head_body,
        grid=(L // RC,),
        in_specs=[
            pl.BlockSpec((RC, DIM), lambda r: (r, 0)),
            pl.BlockSpec((DIM, DIM), lambda r: (0, 0)),
        ],
        out_specs=pl.BlockSpec((RC, DIM), lambda r: (r, 0)),
        out_shape=jax.ShapeDtypeStruct((L, DIM), _F32),
    )(h, w_head)


def kernel(x, params):
    h = x.reshape(L, DIM)
    for i in range(len(params['blocks'])):
        h = _mamba_block(h, params['blocks'][i])
        h = _moe_block(h, params['moes'][i])
    h = _head(h, params['W_head'])
    return h.reshape(x.shape)
